# Initial kernel scaffold; baseline (speedup 1.0000x reference)
import jax
import jax.numpy as jnp
from jax.experimental import pallas as pl


def kernel(vocab_dists, attn_dists, p_gens, enc_batch_extend_vocab):
    raise NotImplementedError("write your pallas kernel here")


# speedup vs baseline: 20.2125x; 20.2125x over previous
import jax
import jax.numpy as jnp
from jax import lax
from jax.experimental import pallas as pl
from jax.experimental.pallas import tpu as pltpu
from jax.experimental.pallas import tpu_sc as plsc

T = 4
B = 128
VOCAB = 100000
ATTN = 200
OOV = 100
VEXT = VOCAB + OOV
ROWS = T * B
LANES = 16
NW = 32
NBANDS = ROWS // 8
BPW = NBANDS // NW

VTILE_FULL = VOCAB // 128
VTILE_REM = VOCAB % 128
OTILES = (VEXT + 127) // 128
OTILE_REM = VEXT % 128

NT = 112
NSEG = 6
NT_LAST = OTILES - NSEG * NT
APAD = 208


def _sc_body(vocab_hbm, attn_hbm, pg_hbm, idx_hbm, out_hbm,
             buf, iv2, av2, pgv, sem):
    wid = lax.axis_index("s") * 2 + lax.axis_index("c")
    lanes = lax.iota(jnp.int32, LANES)
    zf = jnp.zeros((LANES,), jnp.float32)
    zi = jnp.zeros((LANES,), jnp.int32)

    pltpu.sync_copy(pg_hbm, pgv.at[pl.ds(0, ROWS)])
    for r in range(8):
        iv2[pl.ds(r * APAD + 192, LANES)] = zi
        av2[pl.ds(r * APAD + 192, LANES)] = zf

    def in_tile(g, t, tg):
        return (vocab_hbm.at[pl.ds(g * 8, 8), pl.ds(tg * 128, 128)],
                buf.at[t])

    def out_tile(g, t, tg):
        return (buf.at[t],
                out_hbm.at[pl.ds(g * 8, 8), pl.ds(tg * 128, 128)])

    def mul_seg(g, nt):
        def body(t, c):
            for r in range(8):
                pgwin = pgv[pl.ds(g * 8 + r, LANES)]
                pgvec = zf + pgwin[0]
                for j in range(8):
                    buf[t, r, pl.ds(j * LANES, LANES)] = (
                        buf[t, r, pl.ds(j * LANES, LANES)] * pgvec)
            return c
        lax.fori_loop(0, nt, body, 0)

    def scatter_seg(g, t0, nt):
        def rbody(r, c):
            pgwin = pgv[pl.ds(g * 8 + r, LANES)]
            omg = jnp.ones((LANES,), jnp.float32) - (zf + pgwin[0])
            r16 = zi + r

            def cbody(cc, c2):
                ivc = iv2[pl.ds(r * APAD + cc * LANES, LANES)]
                vals = av2[pl.ds(r * APAD + cc * LANES, LANES)] * omg
                tloc = lax.shift_right_logical(ivc, 7) - t0
                cl = lax.bitwise_and(ivc, 127)
                valid = (tloc >= 0) & (tloc < nt)
                for lane in range(LANES):
                    plsc.addupdate_scatter(
                        buf, [tloc, r16, cl], vals,
                        mask=valid & (lanes == lane))
                return c2
            lax.fori_loop(0, APAD // LANES, cbody, 0)
            return c
        lax.fori_loop(0, 8, rbody, 0)

    for i in range(BPW):
        g = wid * BPW + i
        row0 = g * 8
        b0 = lax.rem(row0, B)

        for r in range(8):
            pltpu.sync_copy(idx_hbm.at[pl.ds((b0 + r) * ATTN, 104)],
                            iv2.at[pl.ds(r * APAD, 104)])
            pltpu.sync_copy(idx_hbm.at[pl.ds((b0 + r) * ATTN + 104, 96)],
                            iv2.at[pl.ds(r * APAD + 104, 96)])
            pltpu.sync_copy(attn_hbm.at[pl.ds((row0 + r) * ATTN, 104)],
                            av2.at[pl.ds(r * APAD, 104)])
            pltpu.sync_copy(attn_hbm.at[pl.ds((row0 + r) * ATTN + 104, 96)],
                            av2.at[pl.ds(r * APAD + 104, 96)])

        def seg_body(s, c):
            t0 = s * NT

            def fire(t, c2):
                pltpu.async_copy(*in_tile(g, t, t0 + t), sem)
                return c2
            lax.fori_loop(0, NT, fire, 0)

            def drain(t, c2):
                pltpu.make_async_copy(*in_tile(g, t, t0 + t), sem).wait()
                return c2
            lax.fori_loop(0, NT, drain, 0)

            mul_seg(g, NT)
            scatter_seg(g, t0, NT)

            def ofire(t, c2):
                pltpu.async_copy(*out_tile(g, t, t0 + t), sem)
                return c2
            lax.fori_loop(0, NT, ofire, 0)

            def odrain(t, c2):
                pltpu.make_async_copy(*out_tile(g, t, t0 + t), sem).wait()
                return c2
            lax.fori_loop(0, NT, odrain, 0)
            return c
        lax.fori_loop(0, NSEG, seg_body, 0)

        t0 = NSEG * NT
        nfull = VTILE_FULL - t0

        def lfire(t, c2):
            pltpu.async_copy(*in_tile(g, t, t0 + t), sem)
            return c2
        lax.fori_loop(0, nfull, lfire, 0)
        for r in range(8):
            pltpu.async_copy(
                vocab_hbm.at[row0 + r, pl.ds(VTILE_FULL * 128, VTILE_REM)],
                buf.at[nfull, r, pl.ds(0, VTILE_REM)], sem)

        def ldrain(t, c2):
            pltpu.make_async_copy(*in_tile(g, t, t0 + t), sem).wait()
            return c2
        lax.fori_loop(0, nfull, ldrain, 0)
        for r in range(8):
            pltpu.make_async_copy(
                vocab_hbm.at[row0 + r, pl.ds(VTILE_FULL * 128, VTILE_REM)],
                buf.at[nfull, r, pl.ds(0, VTILE_REM)], sem).wait()

        for r in range(8):
            for j in range(VTILE_REM // LANES, 8):
                buf[nfull, r, pl.ds(j * LANES, LANES)] = zf
            for j in range(8):
                buf[nfull + 1, r, pl.ds(j * LANES, LANES)] = zf

        mul_seg(g, NT_LAST)
        scatter_seg(g, t0, NT_LAST)

        def lofire(t, c2):
            pltpu.async_copy(*out_tile(g, t, t0 + t), sem)
            return c2
        lax.fori_loop(0, nfull + 1, lofire, 0)
        for r in range(8):
            pltpu.async_copy(
                buf.at[nfull + 1, r, pl.ds(0, OTILE_REM)],
                out_hbm.at[row0 + r, pl.ds((OTILES - 1) * 128, OTILE_REM)],
                sem)

        def lodrain(t, c2):
            pltpu.make_async_copy(*out_tile(g, t, t0 + t), sem).wait()
            return c2
        lax.fori_loop(0, nfull + 1, lodrain, 0)
        for r in range(8):
            pltpu.make_async_copy(
                buf.at[nfull + 1, r, pl.ds(0, OTILE_REM)],
                out_hbm.at[row0 + r, pl.ds((OTILES - 1) * 128, OTILE_REM)],
                sem).wait()


@jax.jit
def _final_dist(vocab_dists, attn_dists, p_gens, enc_batch_extend_vocab):
    vocab2 = vocab_dists.reshape(ROWS, VOCAB)
    attn1 = attn_dists.reshape(ROWS * ATTN)
    pg1 = p_gens.reshape(ROWS)
    idx1 = enc_batch_extend_vocab.reshape(B * ATTN)

    mesh = plsc.VectorSubcoreMesh(core_axis_name="c", subcore_axis_name="s")
    run = pl.kernel(
        _sc_body,
        out_type=jax.ShapeDtypeStruct((ROWS, VEXT), jnp.float32),
        mesh=mesh,
        compiler_params=pltpu.CompilerParams(needs_layout_passes=False),
        scratch_types=[
            pltpu.VMEM((NT, 8, 128), jnp.float32),
            pltpu.VMEM((8 * APAD,), jnp.int32),
            pltpu.VMEM((8 * APAD,), jnp.float32),
            pltpu.VMEM((ROWS + LANES,), jnp.float32),
            pltpu.SemaphoreType.DMA,
        ],
    )
    out2 = run(vocab2, attn1, pg1, idx1)
    return out2.reshape(T, B, VEXT)


def kernel(vocab_dists, attn_dists, p_gens, enc_batch_extend_vocab):
    return _final_dist(vocab_dists, attn_dists, p_gens,
                       enc_batch_extend_vocab)
